# baseline (speedup 1.0000x reference)
import functools

import jax
import jax.numpy as jnp
from jax import lax
from jax.experimental import pallas as pl
from jax.experimental.pallas import tpu as pltpu
from jax.experimental.pallas import tpu_sc as plsc

N = 10000
D = 128
TD = 768
E = 320000
EPS = 1e-5

NC = 2
NS = 16
NW = NC * NS
L = 16

NACC = 10240

K = 128
NCH = 80
EP = K * NCH
E2 = NW * EP
PADE = E2 - E

RPT = NACC // NS
SR = 128

BN = 400
NBLK = N // BN

def _deg_body(dst_r, out, dst_all_v, ones_v, acc, semA, semB):
    c = lax.axis_index("c")
    s = lax.axis_index("s")
    w = c * NS + s
    zeros16 = jnp.zeros((L,), jnp.float32)
    ones16 = jnp.ones((L,), jnp.float32)

    def fill0(i, carry):
        for j in range(D // L):
            ones_v[i, pl.ds(j * L, L)] = zeros16
        return carry

    lax.fori_loop(0, K, fill0, 0)
    for t in range(RPT // K):
        pltpu.sync_copy(ones_v, acc.at[pl.ds(s * RPT + t * K, K)])

    def fill1(i, carry):
        for j in range(D // L):
            ones_v[i, pl.ds(j * L, L)] = ones16
        return carry

    lax.fori_loop(0, K, fill1, 0)
    pltpu.sync_copy(dst_r.at[w], dst_all_v)
    plsc.subcore_barrier()

    def chunk(p, carry):
        i0 = 2 * p
        dA = pltpu.async_copy(ones_v, acc.at[dst_all_v.at[i0]], semA, add=True)
        dB = pltpu.async_copy(ones_v, acc.at[dst_all_v.at[i0 + 1]], semB,
                              add=True)
        dA.wait()
        dB.wait()
        return carry

    lax.fori_loop(0, NCH // 2, chunk, 0)
    plsc.subcore_barrier()
    r0 = s * RPT
    pltpu.sync_copy(acc.at[pl.ds(r0, RPT)], out.at[c, pl.ds(r0, RPT)])


def _prop_body(h, src_flat, dst_flat, out, src_v, dstA, dstB, rowsA, rowsB,
               acc, semA, semB):
    c = lax.axis_index("c")
    s = lax.axis_index("s")
    w = c * NS + s

    zeros16 = jnp.zeros((L,), jnp.float32)

    def zero(i, carry):
        for j in range(D // L):
            rowsA[i, pl.ds(j * L, L)] = zeros16
        return carry

    lax.fori_loop(0, K, zero, 0)
    for t in range(RPT // K):
        pltpu.sync_copy(rowsA, acc.at[pl.ds(s * RPT + t * K, K)])

    pltpu.sync_copy(src_flat.at[pl.ds(w * EP, EP)], src_v)

    plsc.subcore_barrier()

    def g_desc(i, buf, sem):
        return pltpu.make_async_copy(h.at[src_v.at[pl.ds(i * K, K)]], buf, sem)

    base_w = w * EP
    pltpu.sync_copy(dst_flat.at[pl.ds(base_w, K)], dstA.at[0])
    pltpu.async_copy(h.at[src_v.at[pl.ds(0, K)]], rowsA, semA)

    def chunk(p, carry):
        i0 = 2 * p
        i1 = i0 + 1
        pltpu.async_copy(h.at[src_v.at[pl.ds(i1 * K, K)]], rowsB, semB)
        pltpu.sync_copy(dst_flat.at[pl.ds(base_w + i1 * K, K)], dstB.at[0])
        g_desc(i0, rowsA, semA).wait()
        pltpu.sync_copy(rowsA, acc.at[dstA.at[0]], add=True)

        @pl.when(p < NCH // 2 - 1)
        def _():
            pltpu.async_copy(h.at[src_v.at[pl.ds((i0 + 2) * K, K)]], rowsA,
                             semA)
            pltpu.sync_copy(dst_flat.at[pl.ds(base_w + (i0 + 2) * K, K)],
                            dstA.at[0])

        g_desc(i1, rowsB, semB).wait()
        pltpu.sync_copy(rowsB, acc.at[dstB.at[0]], add=True)
        return carry

    lax.fori_loop(0, NCH // 2, chunk, 0)

    plsc.subcore_barrier()

    r0 = s * RPT
    pltpu.sync_copy(acc.at[pl.ds(r0, RPT)], out.at[c, pl.ds(r0, RPT)])


@functools.cache
def _sc_kernels():
    mesh = plsc.VectorSubcoreMesh(core_axis_name="c", subcore_axis_name="s",
                                  num_cores=NC, num_subcores=NS)
    deg = pl.kernel(
        _deg_body,
        out_type=jax.ShapeDtypeStruct((NC, NACC, D), jnp.float32),
        mesh=mesh,
        scratch_types=[
            pltpu.VMEM((NCH, K), jnp.int32),
            pltpu.VMEM((K, D), jnp.float32),
            pltpu.VMEM_SHARED((NACC, D), jnp.float32),
            pltpu.SemaphoreType.DMA,
            pltpu.SemaphoreType.DMA,
        ],
    )
    prop = pl.kernel(
        _prop_body,
        out_type=jax.ShapeDtypeStruct((NC, NACC, D), jnp.float32),
        mesh=mesh,
        scratch_types=[
            pltpu.VMEM((EP,), jnp.int32),
            pltpu.VMEM((1, K), jnp.int32),
            pltpu.VMEM((1, K), jnp.int32),
            pltpu.VMEM((K, D), jnp.float32),
            pltpu.VMEM((K, D), jnp.float32),
            pltpu.VMEM_SHARED((NACC, D), jnp.float32),
            pltpu.SemaphoreType.DMA,
            pltpu.SemaphoreType.DMA,
        ],
    )
    return deg, prop


def _stats_body(x_ref, o_ref):
    i = pl.program_id(0)
    xb = x_ref[...]
    s1 = jnp.sum(xb, axis=0, keepdims=True)
    s2 = jnp.sum(xb * xb, axis=0, keepdims=True)
    upd = jnp.concatenate([s1, s2, jnp.zeros((6, D), jnp.float32)], axis=0)

    @pl.when(i == 0)
    def _():
        o_ref[...] = upd

    @pl.when(i > 0)
    def _():
        o_ref[...] = o_ref[...] + upd


def _stats(x):
    return pl.pallas_call(
        _stats_body,
        grid=(NBLK,),
        in_specs=[pl.BlockSpec((BN, D), lambda i: (i, 0))],
        out_specs=pl.BlockSpec((8, D), lambda i: (0, 0)),
        out_shape=jax.ShapeDtypeStruct((8, D), jnp.float32),
    )(x)


def _dis_of(degp_ref):
    deg = degp_ref[0, :, 0:1] + degp_ref[1, :, 0:1] + 1.0
    return lax.rsqrt(deg)


_DEG_SPEC = lambda: pl.BlockSpec((NC, BN, D), lambda i: (0, i, 0))


def _apply_body(x_ref, degp_ref, sc_ref, o_ref):
    dis = _dis_of(degp_ref)
    a = sc_ref[0:1, :]
    b = sc_ref[1:2, :]
    o_ref[...] = dis * (a * x_ref[...] + b)


def _apply(x, degp, scal):
    return pl.pallas_call(
        _apply_body,
        grid=(NBLK,),
        in_specs=[
            pl.BlockSpec((BN, D), lambda i: (i, 0)),
            _DEG_SPEC(),
            pl.BlockSpec((8, D), lambda i: (0, 0)),
        ],
        out_specs=pl.BlockSpec((BN, D), lambda i: (i, 0)),
        out_shape=jax.ShapeDtypeStruct((N, D), jnp.float32),
    )(x, degp, scal)


def _layer1_body(p_ref, hs_ref, degp_ref, w_ref, b_ref, o_ref):
    dis = _dis_of(degp_ref)
    z = dis * (p_ref[0] + p_ref[1] + hs_ref[...])
    y = jnp.dot(z, w_ref[...], preferred_element_type=jnp.float32)
    y = jnp.maximum(y + b_ref[0:1, :], 0.0)
    o_ref[...] = dis * y


def _layer1(parts, hs, degp, Wt, b8):
    return pl.pallas_call(
        _layer1_body,
        grid=(NBLK,),
        in_specs=[
            pl.BlockSpec((NC, BN, D), lambda i: (0, i, 0)),
            pl.BlockSpec((BN, D), lambda i: (i, 0)),
            _DEG_SPEC(),
            pl.BlockSpec((D, D), lambda i: (0, 0)),
            pl.BlockSpec((8, D), lambda i: (0, 0)),
        ],
        out_specs=pl.BlockSpec((BN, D), lambda i: (i, 0)),
        out_shape=jax.ShapeDtypeStruct((N, D), jnp.float32),
    )(parts, hs, degp, Wt, b8)


def _head_body(p_ref, hs_ref, degp_ref, w2_ref, b2_ref, wp1_ref, bp1_ref,
               wp2_ref, bp2_ref, o_ref):
    dis = _dis_of(degp_ref)
    z = dis * (p_ref[0] + p_ref[1] + hs_ref[...])
    h2 = jnp.dot(z, w2_ref[...], preferred_element_type=jnp.float32)
    h2 = jnp.maximum(h2 + b2_ref[0:1, :], 0.0)
    q = jnp.dot(h2, wp1_ref[...], preferred_element_type=jnp.float32)
    q = q + bp1_ref[0:1, :]
    g = 0.5 * q * (1.0 + lax.erf(q * 0.7071067811865476))
    o = jnp.dot(g, wp2_ref[...], preferred_element_type=jnp.float32)
    o_ref[...] = o + bp2_ref[0:1, :]


def _head(parts, hs, degp, W2t, b28, Wp1t, bp18, Wp2t, bp28):
    return pl.pallas_call(
        _head_body,
        grid=(NBLK,),
        in_specs=[
            pl.BlockSpec((NC, BN, D), lambda i: (0, i, 0)),
            pl.BlockSpec((BN, D), lambda i: (i, 0)),
            _DEG_SPEC(),
            pl.BlockSpec((D, D), lambda i: (0, 0)),
            pl.BlockSpec((8, D), lambda i: (0, 0)),
            pl.BlockSpec((D, TD), lambda i: (0, 0)),
            pl.BlockSpec((8, TD), lambda i: (0, 0)),
            pl.BlockSpec((TD, TD), lambda i: (0, 0)),
            pl.BlockSpec((8, TD), lambda i: (0, 0)),
        ],
        out_specs=pl.BlockSpec((BN, TD), lambda i: (i, 0)),
        out_shape=jax.ShapeDtypeStruct((N, TD), jnp.float32),
    )(parts, hs, degp, W2t, b28, Wp1t, bp18, Wp2t, bp28)


def _pad8(v):
    return jnp.broadcast_to(v[None, :], (8, v.shape[0]))


def kernel(x, edge_index, gn_weight, gn_bias, gn_mean_scale, W1, b1, W2, b2,
           Wp1, bp1, Wp2, bp2):
    deg_sc, prop_sc = _sc_kernels()
    src_flat = jnp.concatenate(
        [edge_index[0], jnp.zeros((PADE,), jnp.int32)])
    dst_flat = jnp.concatenate(
        [edge_index[1], jnp.full((PADE,), N, jnp.int32)])
    dst_r = dst_flat.reshape(NW, NCH, K)

    degp = deg_sc(dst_r)

    sums = _stats(x)
    mean = sums[0:1, :] / N
    ex2 = sums[1:2, :] / N
    ms = gn_mean_scale[None, :]
    var = ex2 - (2.0 * ms - ms * ms) * mean * mean
    a = gn_weight[None, :] * lax.rsqrt(var + EPS)
    c = gn_bias[None, :] - a * ms * mean
    scal = jnp.concatenate([a, c, jnp.zeros((6, D), jnp.float32)], axis=0)

    h0s = _apply(x, degp, scal)
    p1 = prop_sc(h0s, src_flat, dst_flat)
    h1s = _layer1(p1, h0s, degp, W1.T, _pad8(b1))
    p2 = prop_sc(h1s, src_flat, dst_flat)
    out = _head(p2, h1s, degp, W2.T, _pad8(b2), Wp1.T, _pad8(bp1),
                Wp2.T, _pad8(bp2))
    return out


# speedup vs baseline: 1.0004x; 1.0004x over previous
import functools

import jax
import jax.numpy as jnp
from jax import lax
from jax.experimental import pallas as pl
from jax.experimental.pallas import tpu as pltpu
from jax.experimental.pallas import tpu_sc as plsc

N = 10000
D = 128
TD = 768
E = 320000
EPS = 1e-5

NC = 2
NS = 16
NW = NC * NS
L = 16

NACC = 10240

K = 128
NCH = 80
EP = K * NCH
E2 = NW * EP
PADE = E2 - E

RPT = NACC // NS
SR = 128

BN = 400
NBLK = N // BN

def _deg_body(dst_r, out, dst_all_v, ones_v, acc, semA, semB):
    c = lax.axis_index("c")
    s = lax.axis_index("s")
    w = c * NS + s
    zeros16 = jnp.zeros((L,), jnp.float32)
    ones16 = jnp.ones((L,), jnp.float32)

    def fill0(i, carry):
        for j in range(D // L):
            ones_v[i, pl.ds(j * L, L)] = zeros16
        return carry

    lax.fori_loop(0, K, fill0, 0)
    for t in range(RPT // K):
        pltpu.sync_copy(ones_v, acc.at[pl.ds(s * RPT + t * K, K)])

    def fill1(i, carry):
        for j in range(D // L):
            ones_v[i, pl.ds(j * L, L)] = ones16
        return carry

    lax.fori_loop(0, K, fill1, 0)
    pltpu.sync_copy(dst_r.at[w], dst_all_v)
    plsc.subcore_barrier()

    def chunk(p, carry):
        i0 = 2 * p
        dA = pltpu.async_copy(ones_v, acc.at[dst_all_v.at[i0]], semA, add=True)
        dB = pltpu.async_copy(ones_v, acc.at[dst_all_v.at[i0 + 1]], semB,
                              add=True)
        dA.wait()
        dB.wait()
        return carry

    lax.fori_loop(0, NCH // 2, chunk, 0)
    plsc.subcore_barrier()
    r0 = s * RPT
    pltpu.sync_copy(acc.at[pl.ds(r0, RPT)], out.at[c, pl.ds(r0, RPT)])


def _prop_body(h, src_flat, dst_flat, out, src_v, dstA, dstB, rowsA, rowsB,
               acc, semA, semB):
    c = lax.axis_index("c")
    s = lax.axis_index("s")
    w = c * NS + s

    zeros16 = jnp.zeros((L,), jnp.float32)

    def zero(i, carry):
        for j in range(D // L):
            rowsA[i, pl.ds(j * L, L)] = zeros16
        return carry

    lax.fori_loop(0, K, zero, 0)
    for t in range(RPT // K):
        pltpu.sync_copy(rowsA, acc.at[pl.ds(s * RPT + t * K, K)])

    pltpu.sync_copy(src_flat.at[pl.ds(w * EP, EP)], src_v)

    plsc.subcore_barrier()

    def g_desc(i, buf, sem):
        return pltpu.make_async_copy(h.at[src_v.at[pl.ds(i * K, K)]], buf, sem)

    base_w = w * EP
    pltpu.sync_copy(dst_flat.at[pl.ds(base_w, K)], dstA.at[0])
    pltpu.async_copy(h.at[src_v.at[pl.ds(0, K)]], rowsA, semA)

    def chunk(p, carry):
        i0 = 2 * p
        i1 = i0 + 1
        pltpu.async_copy(h.at[src_v.at[pl.ds(i1 * K, K)]], rowsB, semB)
        pltpu.sync_copy(dst_flat.at[pl.ds(base_w + i1 * K, K)], dstB.at[0])
        g_desc(i0, rowsA, semA).wait()
        pltpu.sync_copy(rowsA, acc.at[dstA.at[0]], add=True)

        @pl.when(p < NCH // 2 - 1)
        def _():
            pltpu.async_copy(h.at[src_v.at[pl.ds((i0 + 2) * K, K)]], rowsA,
                             semA)
            pltpu.sync_copy(dst_flat.at[pl.ds(base_w + (i0 + 2) * K, K)],
                            dstA.at[0])

        g_desc(i1, rowsB, semB).wait()
        pltpu.sync_copy(rowsB, acc.at[dstB.at[0]], add=True)
        return carry

    lax.fori_loop(0, NCH // 2, chunk, 0)

    plsc.subcore_barrier()

    r0 = s * RPT
    pltpu.sync_copy(acc.at[pl.ds(r0, RPT)], out.at[c, pl.ds(r0, RPT)])


@functools.cache
def _sc_kernels():
    mesh = plsc.VectorSubcoreMesh(core_axis_name="c", subcore_axis_name="s",
                                  num_cores=NC, num_subcores=NS)
    deg = pl.kernel(
        _deg_body,
        out_type=jax.ShapeDtypeStruct((NC, NACC, D), jnp.float32),
        mesh=mesh,
        scratch_types=[
            pltpu.VMEM((NCH, K), jnp.int32),
            pltpu.VMEM((K, D), jnp.float32),
            pltpu.VMEM_SHARED((NACC, D), jnp.float32),
            pltpu.SemaphoreType.DMA,
            pltpu.SemaphoreType.DMA,
        ],
    )
    prop = pl.kernel(
        _prop_body,
        out_type=jax.ShapeDtypeStruct((NC, NACC, D), jnp.float32),
        mesh=mesh,
        scratch_types=[
            pltpu.VMEM((EP,), jnp.int32),
            pltpu.VMEM((1, K), jnp.int32),
            pltpu.VMEM((1, K), jnp.int32),
            pltpu.VMEM((K, D), jnp.float32),
            pltpu.VMEM((K, D), jnp.float32),
            pltpu.VMEM_SHARED((NACC, D), jnp.float32),
            pltpu.SemaphoreType.DMA,
            pltpu.SemaphoreType.DMA,
        ],
    )
    return deg, prop


def _stats_body(x_ref, o_ref):
    i = pl.program_id(0)
    xb = x_ref[...]
    s1 = jnp.sum(xb, axis=0, keepdims=True)
    s2 = jnp.sum(xb * xb, axis=0, keepdims=True)
    upd = jnp.concatenate([s1, s2, jnp.zeros((6, D), jnp.float32)], axis=0)

    @pl.when(i == 0)
    def _():
        o_ref[...] = upd

    @pl.when(i > 0)
    def _():
        o_ref[...] = o_ref[...] + upd


def _stats(x):
    return pl.pallas_call(
        _stats_body,
        grid=(NBLK,),
        in_specs=[pl.BlockSpec((BN, D), lambda i: (i, 0))],
        out_specs=pl.BlockSpec((8, D), lambda i: (0, 0)),
        out_shape=jax.ShapeDtypeStruct((8, D), jnp.float32),
    )(x)


def _dis_of(degp_ref):
    deg = degp_ref[0, :, 0:1] + degp_ref[1, :, 0:1] + 1.0
    return lax.rsqrt(deg)


_DEG_SPEC = lambda: pl.BlockSpec((NC, BN, D), lambda i: (0, i, 0))


def _apply_body(x_ref, degp_ref, sc_ref, o_ref):
    dis = _dis_of(degp_ref)
    a = sc_ref[0:1, :]
    b = sc_ref[1:2, :]
    o_ref[...] = dis * (a * x_ref[...] + b)


def _apply(x, degp, scal):
    return pl.pallas_call(
        _apply_body,
        grid=(NBLK,),
        in_specs=[
            pl.BlockSpec((BN, D), lambda i: (i, 0)),
            _DEG_SPEC(),
            pl.BlockSpec((8, D), lambda i: (0, 0)),
        ],
        out_specs=pl.BlockSpec((BN, D), lambda i: (i, 0)),
        out_shape=jax.ShapeDtypeStruct((N, D), jnp.float32),
    )(x, degp, scal)


def _layer1_body(p_ref, hs_ref, degp_ref, w_ref, b_ref, o_ref):
    dis = _dis_of(degp_ref)
    z = dis * (p_ref[0] + p_ref[1] + hs_ref[...])
    y = jnp.dot(z, w_ref[...], preferred_element_type=jnp.float32)
    y = jnp.maximum(y + b_ref[0:1, :], 0.0)
    o_ref[...] = dis * y


def _layer1(parts, hs, degp, Wt, b8):
    return pl.pallas_call(
        _layer1_body,
        grid=(NBLK,),
        in_specs=[
            pl.BlockSpec((NC, BN, D), lambda i: (0, i, 0)),
            pl.BlockSpec((BN, D), lambda i: (i, 0)),
            _DEG_SPEC(),
            pl.BlockSpec((D, D), lambda i: (0, 0)),
            pl.BlockSpec((8, D), lambda i: (0, 0)),
        ],
        out_specs=pl.BlockSpec((BN, D), lambda i: (i, 0)),
        out_shape=jax.ShapeDtypeStruct((N, D), jnp.float32),
    )(parts, hs, degp, Wt, b8)


def _head_body(p_ref, hs_ref, degp_ref, w2_ref, b2_ref, wp1_ref, bp1_ref,
               wp2_ref, bp2_ref, o_ref):
    dis = _dis_of(degp_ref)
    z = dis * (p_ref[0] + p_ref[1] + hs_ref[...])
    h2 = jnp.dot(z, w2_ref[...], preferred_element_type=jnp.float32)
    h2 = jnp.maximum(h2 + b2_ref[0:1, :], 0.0)
    q = jnp.dot(h2, wp1_ref[...], preferred_element_type=jnp.float32)
    q = q + bp1_ref[0:1, :]
    g = 0.5 * q * (1.0 + lax.erf(q * 0.7071067811865476))
    o = jnp.dot(g, wp2_ref[...], preferred_element_type=jnp.float32)
    o_ref[...] = o + bp2_ref[0:1, :]


def _head(parts, hs, degp, W2t, b28, Wp1t, bp18, Wp2t, bp28):
    return pl.pallas_call(
        _head_body,
        grid=(NBLK,),
        in_specs=[
            pl.BlockSpec((NC, BN, D), lambda i: (0, i, 0)),
            pl.BlockSpec((BN, D), lambda i: (i, 0)),
            _DEG_SPEC(),
            pl.BlockSpec((D, D), lambda i: (0, 0)),
            pl.BlockSpec((8, D), lambda i: (0, 0)),
            pl.BlockSpec((D, TD), lambda i: (0, 0)),
            pl.BlockSpec((8, TD), lambda i: (0, 0)),
            pl.BlockSpec((TD, TD), lambda i: (0, 0)),
            pl.BlockSpec((8, TD), lambda i: (0, 0)),
        ],
        out_specs=pl.BlockSpec((BN, TD), lambda i: (i, 0)),
        out_shape=jax.ShapeDtypeStruct((N, TD), jnp.float32),
    )(parts, hs, degp, W2t, b28, Wp1t, bp18, Wp2t, bp28)


def _pad8(v):
    return jnp.broadcast_to(v[None, :], (8, v.shape[0]))


def kernel(x, edge_index, gn_weight, gn_bias, gn_mean_scale, W1, b1, W2, b2,
           Wp1, bp1, Wp2, bp2):
    deg_sc, prop_sc = _sc_kernels()
    src_flat = jnp.concatenate(
        [edge_index[0], jnp.zeros((PADE,), jnp.int32)])
    pad_dst = N + (jnp.arange(PADE, dtype=jnp.int32) % (NACC - N))
    dst_flat = jnp.concatenate([edge_index[1], pad_dst])
    dst_r = dst_flat.reshape(NW, NCH, K)

    degp = deg_sc(dst_r)

    sums = _stats(x)
    mean = sums[0:1, :] / N
    ex2 = sums[1:2, :] / N
    ms = gn_mean_scale[None, :]
    var = ex2 - (2.0 * ms - ms * ms) * mean * mean
    a = gn_weight[None, :] * lax.rsqrt(var + EPS)
    c = gn_bias[None, :] - a * ms * mean
    scal = jnp.concatenate([a, c, jnp.zeros((6, D), jnp.float32)], axis=0)

    h0s = _apply(x, degp, scal)
    p1 = prop_sc(h0s, src_flat, dst_flat)
    h1s = _layer1(p1, h0s, degp, W1.T, _pad8(b1))
    p2 = prop_sc(h1s, src_flat, dst_flat)
    out = _head(p2, h1s, degp, W2.T, _pad8(b2), Wp1.T, _pad8(bp1),
                Wp2.T, _pad8(bp2))
    return out


# speedup vs baseline: 2.8340x; 2.8330x over previous
import functools

import jax
import jax.numpy as jnp
from jax import lax
from jax.experimental import pallas as pl
from jax.experimental.pallas import tpu as pltpu
from jax.experimental.pallas import tpu_sc as plsc

N = 10000
D = 128
TD = 768
E = 320000
EPS = 1e-5

NC = 2
NS = 16
NW = NC * NS
L = 16

NACC = 10240

K = 128
NCH = 80
EP = K * NCH
E2 = NW * EP
PADE = E2 - E

RPT = NACC // NS
SR = 128

BN = 400
NBLK = N // BN

def _deg_body(dst_r, out, dst_all_v, ones_v, acc, semA, semB):
    c = lax.axis_index("c")
    s = lax.axis_index("s")
    w = c * NS + s
    zeros16 = jnp.zeros((L,), jnp.float32)
    ones16 = jnp.ones((L,), jnp.float32)

    def fill0(i, carry):
        for j in range(D // L):
            ones_v[i, pl.ds(j * L, L)] = zeros16
        return carry

    lax.fori_loop(0, K, fill0, 0)
    for t in range(RPT // K):
        pltpu.sync_copy(ones_v, acc.at[pl.ds(s * RPT + t * K, K)])

    def fill1(i, carry):
        for j in range(D // L):
            ones_v[i, pl.ds(j * L, L)] = ones16
        return carry

    lax.fori_loop(0, K, fill1, 0)
    pltpu.sync_copy(dst_r.at[w], dst_all_v)
    plsc.subcore_barrier()

    def chunk(p, carry):
        i0 = 2 * p
        dA = pltpu.async_copy(ones_v, acc.at[dst_all_v.at[i0]], semA, add=True)
        dB = pltpu.async_copy(ones_v, acc.at[dst_all_v.at[i0 + 1]], semB,
                              add=True)
        dA.wait()
        dB.wait()
        return carry

    lax.fori_loop(0, NCH // 2, chunk, 0)
    plsc.subcore_barrier()
    r0 = s * RPT
    pltpu.sync_copy(acc.at[pl.ds(r0, RPT)], out.at[c, pl.ds(r0, RPT)])


def _prop_body(h, src_flat, dst_flat, out, src_v, dstA, dstB, rowsA, rowsB,
               acc, semA, semB):
    c = lax.axis_index("c")
    s = lax.axis_index("s")
    w = c * NS + s

    zeros16 = jnp.zeros((L,), jnp.float32)

    def zero(i, carry):
        for j in range(D // L):
            rowsA[i, pl.ds(j * L, L)] = zeros16
        return carry

    lax.fori_loop(0, K, zero, 0)
    for t in range(RPT // K):
        pltpu.sync_copy(rowsA, acc.at[pl.ds(s * RPT + t * K, K)])

    pltpu.sync_copy(src_flat.at[pl.ds(w * EP, EP)], src_v)

    plsc.subcore_barrier()

    def g_desc(i, buf, sem):
        return pltpu.make_async_copy(h.at[src_v.at[pl.ds(i * K, K)]], buf, sem)

    base_w = w * EP
    pltpu.sync_copy(dst_flat.at[pl.ds(base_w, K)], dstA.at[0])
    pltpu.async_copy(h.at[src_v.at[pl.ds(0, K)]], rowsA, semA)

    def chunk(p, carry):
        i0 = 2 * p
        i1 = i0 + 1
        pltpu.async_copy(h.at[src_v.at[pl.ds(i1 * K, K)]], rowsB, semB)
        pltpu.sync_copy(dst_flat.at[pl.ds(base_w + i1 * K, K)], dstB.at[0])
        g_desc(i0, rowsA, semA).wait()
        pltpu.sync_copy(rowsA, acc.at[dstA.at[0]], add=True)

        @pl.when(p < NCH // 2 - 1)
        def _():
            pltpu.async_copy(h.at[src_v.at[pl.ds((i0 + 2) * K, K)]], rowsA,
                             semA)
            pltpu.sync_copy(dst_flat.at[pl.ds(base_w + (i0 + 2) * K, K)],
                            dstA.at[0])

        g_desc(i1, rowsB, semB).wait()
        pltpu.sync_copy(rowsB, acc.at[dstB.at[0]], add=True)
        return carry

    lax.fori_loop(0, NCH // 2, chunk, 0)

    plsc.subcore_barrier()

    r0 = s * RPT
    pltpu.sync_copy(acc.at[pl.ds(r0, RPT)], out.at[c, pl.ds(r0, RPT)])


@functools.cache
def _sc_kernels():
    mesh = plsc.VectorSubcoreMesh(core_axis_name="c", subcore_axis_name="s",
                                  num_cores=NC, num_subcores=NS)
    deg = pl.kernel(
        _deg_body,
        out_type=jax.ShapeDtypeStruct((NC, NACC, D), jnp.float32),
        mesh=mesh,
        scratch_types=[
            pltpu.VMEM((NCH, K), jnp.int32),
            pltpu.VMEM((K, D), jnp.float32),
            pltpu.VMEM_SHARED((NACC, D), jnp.float32),
            pltpu.SemaphoreType.DMA,
            pltpu.SemaphoreType.DMA,
        ],
    )
    prop = pl.kernel(
        _prop_body,
        out_type=jax.ShapeDtypeStruct((NC, NACC, D), jnp.float32),
        mesh=mesh,
        scratch_types=[
            pltpu.VMEM((EP,), jnp.int32),
            pltpu.VMEM((1, K), jnp.int32),
            pltpu.VMEM((1, K), jnp.int32),
            pltpu.VMEM((K, D), jnp.float32),
            pltpu.VMEM((K, D), jnp.float32),
            pltpu.VMEM_SHARED((NACC, D), jnp.float32),
            pltpu.SemaphoreType.DMA,
            pltpu.SemaphoreType.DMA,
        ],
    )
    return deg, prop


def _stats_body(x_ref, o_ref):
    i = pl.program_id(0)
    xb = x_ref[...]
    s1 = jnp.sum(xb, axis=0, keepdims=True)
    s2 = jnp.sum(xb * xb, axis=0, keepdims=True)
    upd = jnp.concatenate([s1, s2, jnp.zeros((6, D), jnp.float32)], axis=0)

    @pl.when(i == 0)
    def _():
        o_ref[...] = upd

    @pl.when(i > 0)
    def _():
        o_ref[...] = o_ref[...] + upd


def _stats(x):
    return pl.pallas_call(
        _stats_body,
        grid=(NBLK,),
        in_specs=[pl.BlockSpec((BN, D), lambda i: (i, 0))],
        out_specs=pl.BlockSpec((8, D), lambda i: (0, 0)),
        out_shape=jax.ShapeDtypeStruct((8, D), jnp.float32),
    )(x)


def _dis_of(degp_ref):
    deg = degp_ref[0, :, 0:1] + degp_ref[1, :, 0:1] + 1.0
    return lax.rsqrt(deg)


_DEG_SPEC = lambda: pl.BlockSpec((NC, BN, D), lambda i: (0, i, 0))


def _apply_body(x_ref, degp_ref, sc_ref, o_ref):
    dis = _dis_of(degp_ref)
    a = sc_ref[0:1, :]
    b = sc_ref[1:2, :]
    o_ref[...] = dis * (a * x_ref[...] + b)


def _apply(x, degp, scal):
    return pl.pallas_call(
        _apply_body,
        grid=(NBLK,),
        in_specs=[
            pl.BlockSpec((BN, D), lambda i: (i, 0)),
            _DEG_SPEC(),
            pl.BlockSpec((8, D), lambda i: (0, 0)),
        ],
        out_specs=pl.BlockSpec((BN, D), lambda i: (i, 0)),
        out_shape=jax.ShapeDtypeStruct((N, D), jnp.float32),
    )(x, degp, scal)


def _layer1_body(p_ref, hs_ref, degp_ref, w_ref, b_ref, o_ref):
    dis = _dis_of(degp_ref)
    z = dis * (p_ref[0] + p_ref[1] + hs_ref[...])
    y = jnp.dot(z, w_ref[...], preferred_element_type=jnp.float32)
    y = jnp.maximum(y + b_ref[0:1, :], 0.0)
    o_ref[...] = dis * y


def _layer1(parts, hs, degp, Wt, b8):
    return pl.pallas_call(
        _layer1_body,
        grid=(NBLK,),
        in_specs=[
            pl.BlockSpec((NC, BN, D), lambda i: (0, i, 0)),
            pl.BlockSpec((BN, D), lambda i: (i, 0)),
            _DEG_SPEC(),
            pl.BlockSpec((D, D), lambda i: (0, 0)),
            pl.BlockSpec((8, D), lambda i: (0, 0)),
        ],
        out_specs=pl.BlockSpec((BN, D), lambda i: (i, 0)),
        out_shape=jax.ShapeDtypeStruct((N, D), jnp.float32),
    )(parts, hs, degp, Wt, b8)


def _head_body(p_ref, hs_ref, degp_ref, w2_ref, b2_ref, wp1_ref, bp1_ref,
               wp2_ref, bp2_ref, o_ref):
    dis = _dis_of(degp_ref)
    z = dis * (p_ref[0] + p_ref[1] + hs_ref[...])
    h2 = jnp.dot(z, w2_ref[...], preferred_element_type=jnp.float32)
    h2 = jnp.maximum(h2 + b2_ref[0:1, :], 0.0)
    q = jnp.dot(h2, wp1_ref[...], preferred_element_type=jnp.float32)
    q = q + bp1_ref[0:1, :]
    g = 0.5 * q * (1.0 + lax.erf(q * 0.7071067811865476))
    o = jnp.dot(g, wp2_ref[...], preferred_element_type=jnp.float32)
    o_ref[...] = o + bp2_ref[0:1, :]


def _head(parts, hs, degp, W2t, b28, Wp1t, bp18, Wp2t, bp28):
    return pl.pallas_call(
        _head_body,
        grid=(NBLK,),
        in_specs=[
            pl.BlockSpec((NC, BN, D), lambda i: (0, i, 0)),
            pl.BlockSpec((BN, D), lambda i: (i, 0)),
            _DEG_SPEC(),
            pl.BlockSpec((D, D), lambda i: (0, 0)),
            pl.BlockSpec((8, D), lambda i: (0, 0)),
            pl.BlockSpec((D, TD), lambda i: (0, 0)),
            pl.BlockSpec((8, TD), lambda i: (0, 0)),
            pl.BlockSpec((TD, TD), lambda i: (0, 0)),
            pl.BlockSpec((8, TD), lambda i: (0, 0)),
        ],
        out_specs=pl.BlockSpec((BN, TD), lambda i: (i, 0)),
        out_shape=jax.ShapeDtypeStruct((N, TD), jnp.float32),
    )(parts, hs, degp, W2t, b28, Wp1t, bp18, Wp2t, bp28)


def _pad8(v):
    return jnp.broadcast_to(v[None, :], (8, v.shape[0]))


def kernel(x, edge_index, gn_weight, gn_bias, gn_mean_scale, W1, b1, W2, b2,
           Wp1, bp1, Wp2, bp2):
    deg_sc, prop_sc = _sc_kernels()
    pad_src = jnp.arange(PADE, dtype=jnp.int32) % N
    src_flat = jnp.concatenate([edge_index[0], pad_src])
    pad_dst = N + (jnp.arange(PADE, dtype=jnp.int32) % (NACC - N))
    dst_flat = jnp.concatenate([edge_index[1], pad_dst])
    dst_r = dst_flat.reshape(NW, NCH, K)

    degp = deg_sc(dst_r)

    sums = _stats(x)
    mean = sums[0:1, :] / N
    ex2 = sums[1:2, :] / N
    ms = gn_mean_scale[None, :]
    var = ex2 - (2.0 * ms - ms * ms) * mean * mean
    a = gn_weight[None, :] * lax.rsqrt(var + EPS)
    c = gn_bias[None, :] - a * ms * mean
    scal = jnp.concatenate([a, c, jnp.zeros((6, D), jnp.float32)], axis=0)

    h0s = _apply(x, degp, scal)
    p1 = prop_sc(h0s, src_flat, dst_flat)
    h1s = _layer1(p1, h0s, degp, W1.T, _pad8(b1))
    p2 = prop_sc(h1s, src_flat, dst_flat)
    out = _head(p2, h1s, degp, W2.T, _pad8(b2), Wp1.T, _pad8(bp1),
                Wp2.T, _pad8(bp2))
    return out


# speedup vs baseline: 2.8341x; 1.0000x over previous
import functools

import jax
import jax.numpy as jnp
from jax import lax
from jax.experimental import pallas as pl
from jax.experimental.pallas import tpu as pltpu
from jax.experimental.pallas import tpu_sc as plsc

N = 10000
D = 128
TD = 768
E = 320000
EPS = 1e-5

NC = 2
NS = 16
NW = NC * NS
L = 16

NACC = 10240

K = 128
NCH = 80
EP = K * NCH
E2 = NW * EP
PADE = E2 - E

RPT = NACC // NS
SR = 128

BN = 400
NBLK = N // BN

def _deg_body(dst_r, out, dst_all_v, ones_v, acc, semA, semB):
    c = lax.axis_index("c")
    s = lax.axis_index("s")
    w = c * NS + s
    zeros16 = jnp.zeros((L,), jnp.float32)
    ones16 = jnp.ones((L,), jnp.float32)

    def fill0(i, carry):
        for j in range(D // L):
            ones_v[i, pl.ds(j * L, L)] = zeros16
        return carry

    lax.fori_loop(0, K, fill0, 0)
    for t in range(RPT // K):
        pltpu.sync_copy(ones_v, acc.at[pl.ds(s * RPT + t * K, K)])

    def fill1(i, carry):
        for j in range(D // L):
            ones_v[i, pl.ds(j * L, L)] = ones16
        return carry

    lax.fori_loop(0, K, fill1, 0)
    pltpu.sync_copy(dst_r.at[w], dst_all_v)
    plsc.subcore_barrier()

    def chunk(p, carry):
        i0 = 2 * p
        dA = pltpu.async_copy(ones_v, acc.at[dst_all_v.at[i0]], semA, add=True)
        dB = pltpu.async_copy(ones_v, acc.at[dst_all_v.at[i0 + 1]], semB,
                              add=True)
        dA.wait()
        dB.wait()
        return carry

    lax.fori_loop(0, NCH // 2, chunk, 0)
    plsc.subcore_barrier()
    r0 = s * RPT
    pltpu.sync_copy(acc.at[pl.ds(r0, RPT)], out.at[c, pl.ds(r0, RPT)])


def _prop_body(h, src_flat, dst_flat, out, src_v, dstA, dstB, rowsA, rowsB,
               acc, semA, semB):
    c = lax.axis_index("c")
    s = lax.axis_index("s")
    w = c * NS + s

    zeros16 = jnp.zeros((L,), jnp.float32)

    def zero(i, carry):
        for j in range(D // L):
            rowsA[i, pl.ds(j * L, L)] = zeros16
        return carry

    lax.fori_loop(0, K, zero, 0)
    for t in range(RPT // K):
        pltpu.sync_copy(rowsA, acc.at[pl.ds(s * RPT + t * K, K)])

    pltpu.sync_copy(src_flat.at[pl.ds(w * EP, EP)], src_v)

    plsc.subcore_barrier()

    def g_desc(i, buf, sem):
        return pltpu.make_async_copy(h.at[src_v.at[pl.ds(i * K, K)]], buf, sem)

    base_w = w * EP
    pltpu.sync_copy(dst_flat.at[pl.ds(base_w, K)], dstA.at[0])
    pltpu.async_copy(h.at[src_v.at[pl.ds(0, K)]], rowsA, semA)

    def chunk(p, carry):
        i0 = 2 * p
        i1 = i0 + 1
        pltpu.async_copy(h.at[src_v.at[pl.ds(i1 * K, K)]], rowsB, semB)
        pltpu.sync_copy(dst_flat.at[pl.ds(base_w + i1 * K, K)], dstB.at[0])
        g_desc(i0, rowsA, semA).wait()
        pltpu.sync_copy(rowsA, acc.at[dstA.at[0]], add=True)

        @pl.when(p < NCH // 2 - 1)
        def _():
            pltpu.async_copy(h.at[src_v.at[pl.ds((i0 + 2) * K, K)]], rowsA,
                             semA)
            pltpu.sync_copy(dst_flat.at[pl.ds(base_w + (i0 + 2) * K, K)],
                            dstA.at[0])

        g_desc(i1, rowsB, semB).wait()
        pltpu.sync_copy(rowsB, acc.at[dstB.at[0]], add=True)
        return carry

    lax.fori_loop(0, NCH // 2, chunk, 0)

    plsc.subcore_barrier()

    r0 = s * RPT
    pltpu.sync_copy(acc.at[pl.ds(r0, RPT)], out.at[c, pl.ds(r0, RPT)])


@functools.cache
def _sc_kernels():
    mesh = plsc.VectorSubcoreMesh(core_axis_name="c", subcore_axis_name="s",
                                  num_cores=NC, num_subcores=NS)
    deg = pl.kernel(
        _deg_body,
        out_type=jax.ShapeDtypeStruct((NC, NACC, D), jnp.float32),
        mesh=mesh,
        scratch_types=[
            pltpu.VMEM((NCH, K), jnp.int32),
            pltpu.VMEM((K, D), jnp.float32),
            pltpu.VMEM_SHARED((NACC, D), jnp.float32),
            pltpu.SemaphoreType.DMA,
            pltpu.SemaphoreType.DMA,
        ],
    )
    prop = pl.kernel(
        _prop_body,
        out_type=jax.ShapeDtypeStruct((NC, NACC, D), jnp.float32),
        mesh=mesh,
        scratch_types=[
            pltpu.VMEM((EP,), jnp.int32),
            pltpu.VMEM((1, K), jnp.int32),
            pltpu.VMEM((1, K), jnp.int32),
            pltpu.VMEM((K, D), jnp.float32),
            pltpu.VMEM((K, D), jnp.float32),
            pltpu.VMEM_SHARED((NACC, D), jnp.float32),
            pltpu.SemaphoreType.DMA,
            pltpu.SemaphoreType.DMA,
        ],
    )
    return deg, prop


def _stats_body(x_ref, o_ref):
    i = pl.program_id(0)
    xb = x_ref[...]
    s1 = jnp.sum(xb, axis=0, keepdims=True)
    s2 = jnp.sum(xb * xb, axis=0, keepdims=True)
    upd = jnp.concatenate([s1, s2, jnp.zeros((6, D), jnp.float32)], axis=0)

    @pl.when(i == 0)
    def _():
        o_ref[...] = upd

    @pl.when(i > 0)
    def _():
        o_ref[...] = o_ref[...] + upd


def _stats(x):
    return pl.pallas_call(
        _stats_body,
        grid=(NBLK,),
        in_specs=[pl.BlockSpec((BN, D), lambda i: (i, 0))],
        out_specs=pl.BlockSpec((8, D), lambda i: (0, 0)),
        out_shape=jax.ShapeDtypeStruct((8, D), jnp.float32),
    )(x)


def _dis_of(degp_ref):
    deg = degp_ref[0, :, 0:1] + degp_ref[1, :, 0:1] + 1.0
    return lax.rsqrt(deg)


_DEG_SPEC = lambda: pl.BlockSpec((NC, BN, D), lambda i: (0, i, 0))


def _apply_body(x_ref, degp_ref, sc_ref, o_ref):
    dis = _dis_of(degp_ref)
    a = sc_ref[0:1, :]
    b = sc_ref[1:2, :]
    o_ref[...] = dis * (a * x_ref[...] + b)


def _apply(x, degp, scal):
    return pl.pallas_call(
        _apply_body,
        grid=(NBLK,),
        in_specs=[
            pl.BlockSpec((BN, D), lambda i: (i, 0)),
            _DEG_SPEC(),
            pl.BlockSpec((8, D), lambda i: (0, 0)),
        ],
        out_specs=pl.BlockSpec((BN, D), lambda i: (i, 0)),
        out_shape=jax.ShapeDtypeStruct((N, D), jnp.float32),
    )(x, degp, scal)


def _layer1_body(p_ref, hs_ref, degp_ref, w_ref, b_ref, o_ref):
    dis = _dis_of(degp_ref)
    z = dis * (p_ref[0] + p_ref[1] + hs_ref[...])
    y = jnp.dot(z.astype(jnp.bfloat16), w_ref[...],
                preferred_element_type=jnp.float32)
    y = jnp.maximum(y + b_ref[0:1, :], 0.0)
    o_ref[...] = dis * y


def _layer1(parts, hs, degp, Wt, b8):
    return pl.pallas_call(
        _layer1_body,
        grid=(NBLK,),
        in_specs=[
            pl.BlockSpec((NC, BN, D), lambda i: (0, i, 0)),
            pl.BlockSpec((BN, D), lambda i: (i, 0)),
            _DEG_SPEC(),
            pl.BlockSpec((D, D), lambda i: (0, 0)),
            pl.BlockSpec((8, D), lambda i: (0, 0)),
        ],
        out_specs=pl.BlockSpec((BN, D), lambda i: (i, 0)),
        out_shape=jax.ShapeDtypeStruct((N, D), jnp.float32),
    )(parts, hs, degp, Wt, b8)


def _head_body(p_ref, hs_ref, degp_ref, w2_ref, b2_ref, wp1_ref, bp1_ref,
               wp2_ref, bp2_ref, o_ref):
    dis = _dis_of(degp_ref)
    z = dis * (p_ref[0] + p_ref[1] + hs_ref[...])
    h2 = jnp.dot(z.astype(jnp.bfloat16), w2_ref[...],
                 preferred_element_type=jnp.float32)
    h2 = jnp.maximum(h2 + b2_ref[0:1, :], 0.0)
    q = jnp.dot(h2.astype(jnp.bfloat16), wp1_ref[...],
                preferred_element_type=jnp.float32)
    q = q + bp1_ref[0:1, :]
    g = 0.5 * q * (1.0 + lax.erf(q * 0.7071067811865476))
    o = jnp.dot(g.astype(jnp.bfloat16), wp2_ref[...],
                preferred_element_type=jnp.float32)
    o_ref[...] = o + bp2_ref[0:1, :]


def _head(parts, hs, degp, W2t, b28, Wp1t, bp18, Wp2t, bp28):
    return pl.pallas_call(
        _head_body,
        grid=(NBLK,),
        in_specs=[
            pl.BlockSpec((NC, BN, D), lambda i: (0, i, 0)),
            pl.BlockSpec((BN, D), lambda i: (i, 0)),
            _DEG_SPEC(),
            pl.BlockSpec((D, D), lambda i: (0, 0)),
            pl.BlockSpec((8, D), lambda i: (0, 0)),
            pl.BlockSpec((D, TD), lambda i: (0, 0)),
            pl.BlockSpec((8, TD), lambda i: (0, 0)),
            pl.BlockSpec((TD, TD), lambda i: (0, 0)),
            pl.BlockSpec((8, TD), lambda i: (0, 0)),
        ],
        out_specs=pl.BlockSpec((BN, TD), lambda i: (i, 0)),
        out_shape=jax.ShapeDtypeStruct((N, TD), jnp.float32),
    )(parts, hs, degp, W2t, b28, Wp1t, bp18, Wp2t, bp28)


def _pad8(v):
    return jnp.broadcast_to(v[None, :], (8, v.shape[0]))


def kernel(x, edge_index, gn_weight, gn_bias, gn_mean_scale, W1, b1, W2, b2,
           Wp1, bp1, Wp2, bp2):
    deg_sc, prop_sc = _sc_kernels()
    pad_src = jnp.arange(PADE, dtype=jnp.int32) % N
    src_flat = jnp.concatenate([edge_index[0], pad_src])
    pad_dst = N + (jnp.arange(PADE, dtype=jnp.int32) % (NACC - N))
    dst_flat = jnp.concatenate([edge_index[1], pad_dst])
    dst_r = dst_flat.reshape(NW, NCH, K)

    degp = deg_sc(dst_r)

    sums = _stats(x)
    mean = sums[0:1, :] / N
    ex2 = sums[1:2, :] / N
    ms = gn_mean_scale[None, :]
    var = ex2 - (2.0 * ms - ms * ms) * mean * mean
    a = gn_weight[None, :] * lax.rsqrt(var + EPS)
    c = gn_bias[None, :] - a * ms * mean
    scal = jnp.concatenate([a, c, jnp.zeros((6, D), jnp.float32)], axis=0)

    h0s = _apply(x, degp, scal)
    p1 = prop_sc(h0s, src_flat, dst_flat)
    bf = jnp.bfloat16
    h1s = _layer1(p1, h0s, degp, W1.T.astype(bf), _pad8(b1))
    p2 = prop_sc(h1s, src_flat, dst_flat)
    out = _head(p2, h1s, degp, W2.T.astype(bf), _pad8(b2),
                Wp1.T.astype(bf), _pad8(bp1), Wp2.T.astype(bf), _pad8(bp2))
    return out
